# baseline (speedup 1.0000x reference)
import jax
import jax.numpy as jnp
from jax import lax
from jax.experimental import pallas as pl
from jax.experimental.pallas import tpu as pltpu
from jax.experimental.pallas import tpu_sc as plsc

_BATCH = 16384
_N_FIELDS = 26
_VOCAB = 100000
_EMB_DIM = 16
_N_CONT = 10
_OUT_W = _N_FIELDS * _EMB_DIM + _N_CONT
_PACK = 8
_TAB_ROWS = _N_FIELDS * _VOCAB // _PACK
_ROW_W = _PACK * _EMB_DIM

_NC = 2
_NS = 16
_LANES = 16
_NW = _NC * _NS
_BPW = _BATCH // _NW
_BANK = 256
_NBANK = _BPW // _BANK
_BGRP = _BANK // _LANES
_CSL = _BPW * _N_CONT // _LANES


def _body(tab_hbm, xcat_hbm, xcont_hbm, cmin_hbm, cmax_hbm, out_hbm,
          xc_v, ri0_v, ri1_v, sh0_v, sh1_v, fc0_v, fc1_v, ext_v,
          cin_v, cm_v, cx_v, sem0, sem1):
    wid = lax.axis_index("s") * _NC + lax.axis_index("c")
    base = wid * _BPW
    iota = lax.iota(jnp.int32, _LANES)

    pltpu.sync_copy(xcat_hbm.at[pl.ds(base, _BPW)], xc_v)

    riv = (ri0_v, ri1_v)
    shv = (sh0_v, sh1_v)
    fcv = (fc0_v, fc1_v)
    sems = (sem0, sem1)

    def cont_path():
        pltpu.sync_copy(xcont_hbm.at[pl.ds(base, _BPW)], cin_v)
        pltpu.sync_copy(cmin_hbm, cm_v)
        pltpu.sync_copy(cmax_hbm, cx_v)

        @pl.loop(0, _CSL)
        def _(j):
            p = j * _LANES + iota
            r = p // _N_CONT
            c = p % _N_CONT
            x = plsc.load_gather(cin_v, [r, c])
            mn = plsc.load_gather(cm_v, [c])
            mx = plsc.load_gather(cx_v, [c])
            plsc.store_scatter(cin_v, [r, c], (x - mn) / (mx - mn))

        pltpu.sync_copy(
            cin_v,
            out_hbm.at[pl.ds(base, _BPW),
                       pl.ds(_N_FIELDS * _EMB_DIM, _N_CONT)])

    def do_field(f):
        fv = f * _VOCAB
        fvec = jnp.full((_LANES,), 0, jnp.int32) + f
        desc = []
        for h in range(_NBANK):
            @pl.loop(0, _BGRP)
            def _(g, h=h):
                rvec = h * _BANK + g * _LANES + iota
                col = plsc.load_gather(xc_v, [rvec, fvec]) + fv
                riv[h][pl.ds(g * _LANES, _LANES)] = col // _PACK
                shv[h][pl.ds(g * _LANES, _LANES)] = (
                    (col % _PACK) * _EMB_DIM)
            desc.append(
                pltpu.async_copy(tab_hbm.at[riv[h]], fcv[h], sems[h]))
        colout = pl.multiple_of(f * _EMB_DIM, _EMB_DIM)
        for h in range(_NBANK):
            desc[h].wait()

            @pl.loop(0, _BGRP)
            def _(g, h=h):
                svec = shv[h][pl.ds(g * _LANES, _LANES)]
                for l in range(_LANES):
                    j = g * _LANES + l
                    ext_v[j] = fcv[h][j, pl.ds(svec[l], _EMB_DIM)]

            pltpu.sync_copy(
                ext_v,
                out_hbm.at[pl.ds(base + h * _BANK, _BANK),
                           pl.ds(colout, _EMB_DIM)])

    do_field(0)
    cont_path()

    @pl.loop(1, _N_FIELDS)
    def _(f):
        do_field(f)


_mesh = plsc.VectorSubcoreMesh(core_axis_name="c", subcore_axis_name="s")

_sc_call = pl.kernel(
    _body,
    out_type=jax.ShapeDtypeStruct((_BATCH, _OUT_W), jnp.float32),
    mesh=_mesh,
    scratch_types=[
        pltpu.VMEM((_BPW, _N_FIELDS), jnp.int32),
        pltpu.VMEM((_BANK,), jnp.int32),
        pltpu.VMEM((_BANK,), jnp.int32),
        pltpu.VMEM((_BANK,), jnp.int32),
        pltpu.VMEM((_BANK,), jnp.int32),
        pltpu.VMEM((_BANK, _ROW_W), jnp.float32),
        pltpu.VMEM((_BANK, _ROW_W), jnp.float32),
        pltpu.VMEM((_BANK, _EMB_DIM), jnp.float32),
        pltpu.VMEM((_BPW, _N_CONT), jnp.float32),
        pltpu.VMEM((_N_CONT,), jnp.float32),
        pltpu.VMEM((_N_CONT,), jnp.float32),
        pltpu.SemaphoreType.DMA,
        pltpu.SemaphoreType.DMA,
    ],
    compiler_params=pltpu.CompilerParams(
        use_tc_tiling_on_sc=False, needs_layout_passes=False),
)


@jax.jit
def kernel(x_cat, x_cont, tables, cont_min, cont_max):
    xcat = x_cat.astype(jnp.int32)
    tab_packed = tables.reshape(_TAB_ROWS, _ROW_W)
    return _sc_call(tab_packed, xcat, x_cont, cont_min, cont_max)


# speedup vs baseline: 1.0641x; 1.0641x over previous
import jax
import jax.numpy as jnp
from jax import lax
from jax.experimental import pallas as pl
from jax.experimental.pallas import tpu as pltpu
from jax.experimental.pallas import tpu_sc as plsc

_BATCH = 16384
_N_FIELDS = 26
_VOCAB = 100000
_EMB_DIM = 16
_N_CONT = 10
_OUT_W = _N_FIELDS * _EMB_DIM + _N_CONT

_NC = 2
_NS = 16
_LANES = 16
_NW = _NC * _NS
_BPW = _BATCH // _NW
_GRP = _BPW // _LANES


def _body(tab_hbm, xcatt_hbm, xcontt_hbm, cmin_hbm, cmax_hbm, outt_hbm,
          xc_v, idx0_v, idx1_v, fc0_v, fc1_v, fct_v,
          cin_v, cm_v, cx_v, sem0, sem1):
    wid = lax.axis_index("s") * _NC + lax.axis_index("c")
    base = wid * _BPW
    iota = lax.iota(jnp.int32, _LANES)

    pltpu.sync_copy(xcatt_hbm.at[:, pl.ds(base, _BPW)], xc_v)

    idxv = (idx0_v, idx1_v)
    fcv = (fc0_v, fc1_v)
    sems = (sem0, sem1)
    desc = [None, None]

    def build_idx(f):
        idxr = idxv[f & 1]
        fvec = jnp.full((_LANES,), f, jnp.int32)

        @pl.loop(0, _GRP)
        def _(g):
            rvec = g * _LANES + iota
            col = plsc.load_gather(xc_v, [fvec, rvec])
            idxr[pl.ds(g * _LANES, _LANES)] = col

    def emit_field(f):
        prev = 1 - (f & 1) if f >= 0 else 0
        fcr = fcv[f & 1]

        @pl.loop(0, _BPW)
        def _(j):
            y = fcr[j]
            plsc.store_scatter(
                fct_v, [iota, jnp.full((_LANES,), 0, jnp.int32) + j], y)

        pltpu.sync_copy(
            fct_v,
            outt_hbm.at[pl.ds(f * _EMB_DIM, _EMB_DIM), pl.ds(base, _BPW)])

    def cont_path():
        pltpu.sync_copy(xcontt_hbm.at[:, pl.ds(base, _BPW)], cin_v)
        pltpu.sync_copy(cmin_hbm, cm_v.at[pl.ds(0, _N_CONT)])
        pltpu.sync_copy(cmax_hbm, cx_v.at[pl.ds(0, _N_CONT)])
        mnv = cm_v[...]
        mxv = cx_v[...]
        for c in range(_N_CONT):
            mn = mnv[c]
            den = mxv[c] - mn

            @pl.loop(0, _GRP)
            def _(g, c=c, mn=mn, den=den):
                s = pl.ds(g * _LANES, _LANES)
                cin_v[c, s] = (cin_v[c, s] - mn) / den

        pltpu.sync_copy(
            cin_v,
            outt_hbm.at[pl.ds(_N_FIELDS * _EMB_DIM, _N_CONT),
                        pl.ds(base, _BPW)])

    for f in range(_N_FIELDS):
        cur = f & 1
        build_idx(f)
        desc[cur] = pltpu.async_copy(tab_hbm.at[f].at[idxv[cur]], fcv[cur],
                                     sems[cur])
        if f == 0:
            cont_path()
        if f >= 1:
            desc[1 - cur].wait()
            emit_field(f - 1)
    desc[1].wait()
    emit_field(_N_FIELDS - 1)


_mesh = plsc.VectorSubcoreMesh(core_axis_name="c", subcore_axis_name="s")

_sc_call = pl.kernel(
    _body,
    out_type=jax.ShapeDtypeStruct((_OUT_W, _BATCH), jnp.float32),
    mesh=_mesh,
    scratch_types=[
        pltpu.VMEM((_N_FIELDS, _BPW), jnp.int32),
        pltpu.VMEM((_BPW,), jnp.int32),
        pltpu.VMEM((_BPW,), jnp.int32),
        pltpu.VMEM((_BPW, _EMB_DIM), jnp.float32),
        pltpu.VMEM((_BPW, _EMB_DIM), jnp.float32),
        pltpu.VMEM((_EMB_DIM, _BPW), jnp.float32),
        pltpu.VMEM((_N_CONT, _BPW), jnp.float32),
        pltpu.VMEM((_LANES,), jnp.float32),
        pltpu.VMEM((_LANES,), jnp.float32),
        pltpu.SemaphoreType.DMA,
        pltpu.SemaphoreType.DMA,
    ],
    compiler_params=pltpu.CompilerParams(
        use_tc_tiling_on_sc=False, needs_layout_passes=False),
)


@jax.jit
def kernel(x_cat, x_cont, tables, cont_min, cont_max):
    xcatt = x_cat.astype(jnp.int32).T
    xcontt = x_cont.T
    outt = _sc_call(tables, xcatt, xcontt, cont_min, cont_max)
    return outt.T


# speedup vs baseline: 1.1235x; 1.0559x over previous
import jax
import jax.numpy as jnp
from jax import lax
from jax.experimental import pallas as pl
from jax.experimental.pallas import tpu as pltpu
from jax.experimental.pallas import tpu_sc as plsc

_BATCH = 16384
_N_FIELDS = 26
_VOCAB = 100000
_EMB_DIM = 16
_N_CONT = 10
_OUT_W = _N_FIELDS * _EMB_DIM + _N_CONT

_NC = 2
_NS = 16
_LANES = 16
_NW = _NC * _NS
_BPW = _BATCH // _NW
_GRP = _BPW // _LANES
_CSL = _BPW * _N_CONT // _LANES


def _body(tab_hbm, xcatt_hbm, xcont_hbm, cmin_hbm, cmax_hbm, out_hbm,
          xc_v, idx0_v, idx1_v, fc0_v, fc1_v,
          cin_v, cout_v, cm_v, cx_v, sem0, sem1):
    wid = lax.axis_index("s") * _NC + lax.axis_index("c")
    base = wid * _BPW
    iota = lax.iota(jnp.int32, _LANES)

    pltpu.sync_copy(xcatt_hbm.at[:, pl.ds(base, _BPW)], xc_v)

    idxv = (idx0_v, idx1_v)
    fcv = (fc0_v, fc1_v)
    sems = (sem0, sem1)
    desc = [None, None]

    def build_idx(f):
        idxr = idxv[f & 1]
        fvec = jnp.full((_LANES,), f, jnp.int32)

        @pl.loop(0, _GRP)
        def _(g):
            rvec = g * _LANES + iota
            col = plsc.load_gather(xc_v, [fvec, rvec])
            idxr[pl.ds(g * _LANES, _LANES)] = col + f * _VOCAB

    def cont_path():
        pltpu.sync_copy(xcont_hbm.at[pl.ds(base, _BPW)], cin_v)
        pltpu.sync_copy(cmin_hbm, cm_v)
        pltpu.sync_copy(cmax_hbm, cx_v)

        @pl.loop(0, _CSL)
        def _(j):
            p = j * _LANES + iota
            r = p // _N_CONT
            c = p % _N_CONT
            x = plsc.load_gather(cin_v, [r, c])
            mn = plsc.load_gather(cm_v, [c])
            mx = plsc.load_gather(cx_v, [c])
            plsc.store_scatter(cout_v, [r, c], (x - mn) / (mx - mn))

        pltpu.sync_copy(
            cout_v,
            out_hbm.at[pl.ds(base, _BPW),
                       pl.ds(_N_FIELDS * _EMB_DIM, _N_CONT)])

    for f in range(_N_FIELDS):
        cur = f & 1
        build_idx(f)
        desc[cur] = pltpu.async_copy(tab_hbm.at[idxv[cur]], fcv[cur],
                                     sems[cur])
        if f == 0:
            cont_path()
        if f >= 1:
            prev = 1 - cur
            desc[prev].wait()
            pltpu.sync_copy(
                fcv[prev],
                out_hbm.at[pl.ds(base, _BPW),
                           pl.ds((f - 1) * _EMB_DIM, _EMB_DIM)])
    desc[1].wait()
    pltpu.sync_copy(
        fcv[1],
        out_hbm.at[pl.ds(base, _BPW),
                   pl.ds((_N_FIELDS - 1) * _EMB_DIM, _EMB_DIM)])


_mesh = plsc.VectorSubcoreMesh(core_axis_name="c", subcore_axis_name="s")

_sc_call = pl.kernel(
    _body,
    out_type=jax.ShapeDtypeStruct((_BATCH, _OUT_W), jnp.float32),
    mesh=_mesh,
    scratch_types=[
        pltpu.VMEM((_N_FIELDS, _BPW), jnp.int32),
        pltpu.VMEM((_BPW,), jnp.int32),
        pltpu.VMEM((_BPW,), jnp.int32),
        pltpu.VMEM((_BPW, _EMB_DIM), jnp.float32),
        pltpu.VMEM((_BPW, _EMB_DIM), jnp.float32),
        pltpu.VMEM((_BPW, _N_CONT), jnp.float32),
        pltpu.VMEM((_BPW, _N_CONT), jnp.float32),
        pltpu.VMEM((_N_CONT,), jnp.float32),
        pltpu.VMEM((_N_CONT,), jnp.float32),
        pltpu.SemaphoreType.DMA,
        pltpu.SemaphoreType.DMA,
    ],
    compiler_params=pltpu.CompilerParams(
        use_tc_tiling_on_sc=False, needs_layout_passes=False),
)


@jax.jit
def kernel(x_cat, x_cont, tables, cont_min, cont_max):
    xcatt = x_cat.astype(jnp.int32).T
    tab_flat = tables.reshape(_N_FIELDS * _VOCAB, _EMB_DIM)
    return _sc_call(tab_flat, xcatt, x_cont, cont_min, cont_max)
